# Initial kernel scaffold; baseline (speedup 1.0000x reference)
import jax
import jax.numpy as jnp
import numpy as np
from jax import lax
from jax.experimental import pallas as pl
from jax.experimental.pallas import tpu as pltpu
from jax.experimental.pallas import tpu_sc as plsc

_VOCAB = 100000
_EMBED = 32
_NUM_TOKENS = 26
_B = 4096
_L = 50

_N = _B * _L * _NUM_TOKENS
_NW = 32
_PER_W = _N // _NW
_BATCH = 1664
_NBATCH = _PER_W // _BATCH
_FIRE = 128
_NFIRE = _BATCH // _FIRE
_GROUPS = _BATCH // 16

assert _PER_W * _NW == _N
assert _NBATCH * _BATCH == _PER_W
assert _PER_W % 208 == 0 and _BATCH % 208 == 0

_OFF = jnp.asarray((np.arange(208) % _NUM_TOKENS) * _VOCAB, dtype=jnp.int32)


def _gather_body(tab_hbm, idx_hbm, off_hbm, out_hbm,
                 idx_v, gidx_v, rows_v, off_v, sem):
    nc = 2
    wid = lax.axis_index("s") * nc + lax.axis_index("c")
    base = wid * _PER_W
    pltpu.sync_copy(off_hbm, off_v)

    def batch(k, carry):
        b0 = base + k * _BATCH
        pltpu.sync_copy(idx_hbm.at[pl.ds(b0, _BATCH)], idx_v)
        for g in range(_GROUPS):
            sl = pl.ds(g * 16, 16)
            gidx_v[sl] = idx_v[sl] + off_v[pl.ds((g % 13) * 16, 16)]
        descs = [
            pltpu.async_copy(
                tab_hbm.at[gidx_v.at[pl.ds(f * _FIRE, _FIRE)]],
                rows_v.at[pl.ds(f * _FIRE, _FIRE)],
                sem,
            )
            for f in range(_NFIRE)
        ]
        for d in descs:
            d.wait()
        pltpu.sync_copy(rows_v, out_hbm.at[pl.ds(b0, _BATCH)])
        return carry

    lax.fori_loop(0, _NBATCH, batch, 0)


@jax.jit
def _flat_gather(tab, idx, off):
    mesh = plsc.VectorSubcoreMesh(core_axis_name="c", subcore_axis_name="s")
    return pl.kernel(
        _gather_body,
        out_type=jax.ShapeDtypeStruct((_N, _EMBED), jnp.float32),
        mesh=mesh,
        scratch_types=[
            pltpu.VMEM((_BATCH,), jnp.int32),
            pltpu.VMEM((_BATCH,), jnp.int32),
            pltpu.VMEM((_BATCH, _EMBED), jnp.float32),
            pltpu.VMEM((208,), jnp.int32),
            pltpu.SemaphoreType.DMA,
        ],
    )(tab, idx, off)


def kernel(x, tables):
    tab = tables.reshape(_NUM_TOKENS * _VOCAB, _EMBED)
    idx = x.reshape(_N)
    out = _flat_gather(tab, idx, _OFF)
    return out.reshape(_B, _L, _NUM_TOKENS * _EMBED)


# speedup vs baseline: 7.9608x; 7.9608x over previous
import jax
import jax.numpy as jnp
import numpy as np
from jax import lax
from jax.experimental import pallas as pl
from jax.experimental.pallas import tpu as pltpu
from jax.experimental.pallas import tpu_sc as plsc

_VOCAB = 100000
_EMBED = 32
_NUM_TOKENS = 26
_B = 4096
_L = 50

_N = _B * _L * _NUM_TOKENS
_NW = 32
_PER_W = _N // _NW
_BATCH = 1664
_NBATCH = _PER_W // _BATCH
_FIRE = 128
_NFIRE = _BATCH // _FIRE
_GROUPS = _BATCH // 16

assert _PER_W * _NW == _N
assert _NBATCH * _BATCH == _PER_W
assert _PER_W % 208 == 0 and _BATCH % 208 == 0

_OFF = np.asarray((np.arange(208) % _NUM_TOKENS) * _VOCAB, dtype=np.int32)


def _gather_body(tab_hbm, idx_hbm, off_hbm, out_hbm,
                 idx_v, gidx_v, rows_v, off_v, sem):
    nc = 2
    wid = lax.axis_index("s") * nc + lax.axis_index("c")
    base = wid * _PER_W
    pltpu.sync_copy(off_hbm, off_v)

    def batch(k, carry):
        b0 = base + k * _BATCH
        pltpu.sync_copy(idx_hbm.at[pl.ds(b0, _BATCH)], idx_v)
        for g in range(_GROUPS):
            sl = pl.ds(g * 16, 16)
            gidx_v[sl] = idx_v[sl] + off_v[pl.ds((g % 13) * 16, 16)]
        descs = [
            pltpu.async_copy(
                tab_hbm.at[gidx_v.at[pl.ds(f * _FIRE, _FIRE)]],
                rows_v.at[pl.ds(f * _FIRE, _FIRE)],
                sem,
            )
            for f in range(_NFIRE)
        ]
        for d in descs:
            d.wait()
        pltpu.sync_copy(rows_v, out_hbm.at[pl.ds(b0, _BATCH)])
        return carry

    lax.fori_loop(0, _NBATCH, batch, 0)


@jax.jit
def _flat_gather(tab, idx, off):
    mesh = plsc.VectorSubcoreMesh(core_axis_name="c", subcore_axis_name="s")
    return pl.kernel(
        _gather_body,
        out_type=jax.ShapeDtypeStruct((_N, _EMBED), jnp.float32),
        mesh=mesh,
        compiler_params=pltpu.CompilerParams(use_tc_tiling_on_sc=False),
        scratch_types=[
            pltpu.VMEM((_BATCH,), jnp.int32),
            pltpu.VMEM((_BATCH,), jnp.int32),
            pltpu.VMEM((_BATCH, _EMBED), jnp.float32),
            pltpu.VMEM((208,), jnp.int32),
            pltpu.SemaphoreType.DMA,
        ],
    )(tab, idx, off)


def kernel(x, tables):
    tab = tables.reshape(_NUM_TOKENS * _VOCAB, _EMBED)
    idx = x.reshape(_N)
    out = _flat_gather(tab, idx, _OFF)
    return out.reshape(_B, _L, _NUM_TOKENS * _EMBED)


# speedup vs baseline: 8.2330x; 1.0342x over previous
import jax
import jax.numpy as jnp
import numpy as np
from jax import lax
from jax.experimental import pallas as pl
from jax.experimental.pallas import tpu as pltpu
from jax.experimental.pallas import tpu_sc as plsc

_VOCAB = 100000
_EMBED = 32
_NUM_TOKENS = 26
_B = 4096
_L = 50

_N = _B * _L * _NUM_TOKENS
_NW = 32
_PER_W = _N // _NW
_BATCH = 1664
_NBATCH = _PER_W // _BATCH
_FIRE = 128
_NFIRE = _BATCH // _FIRE
_GROUPS = _BATCH // 16

assert _PER_W * _NW == _N
assert _NBATCH * _BATCH == _PER_W and _NBATCH % 2 == 0
assert _PER_W % 208 == 0 and _BATCH % 208 == 0

_OFF = np.asarray((np.arange(208) % _NUM_TOKENS) * _VOCAB, dtype=np.int32)


def _gather_body(tab_hbm, idx_hbm, off_hbm, out_hbm,
                 idx0, idx1, gidx0, gidx1, rows0, rows1, off_v,
                 gsem0, gsem1, wsem0, wsem1):
    nc = 2
    wid = lax.axis_index("s") * nc + lax.axis_index("c")
    base = wid * _PER_W
    pltpu.sync_copy(off_hbm, off_v)

    def load_transform(k, idx_b, gidx_b):
        b0 = base + k * _BATCH
        pltpu.sync_copy(idx_hbm.at[pl.ds(b0, _BATCH)], idx_b)
        for g in range(_GROUPS):
            sl = pl.ds(g * 16, 16)
            gidx_b[sl] = idx_b[sl] + off_v[pl.ds((g % 13) * 16, 16)]

    def fire_gather(gidx_b, rows_b, sem):
        for f in range(_NFIRE):
            pltpu.async_copy(
                tab_hbm.at[gidx_b.at[pl.ds(f * _FIRE, _FIRE)]],
                rows_b.at[pl.ds(f * _FIRE, _FIRE)],
                sem,
            )

    def wait_gather(rows_b, sem):
        pltpu.make_async_copy(out_hbm.at[pl.ds(0, _BATCH)], rows_b, sem).wait()

    def fire_write(k, rows_b, sem):
        pltpu.async_copy(rows_b, out_hbm.at[pl.ds(base + k * _BATCH, _BATCH)], sem)

    def wait_write(k, rows_b, sem):
        pltpu.make_async_copy(rows_b, out_hbm.at[pl.ds(base + k * _BATCH, _BATCH)], sem).wait()

    load_transform(0, idx0, gidx0)
    fire_gather(gidx0, rows0, gsem0)
    load_transform(1, idx1, gidx1)
    fire_gather(gidx1, rows1, gsem1)

    def pair(j, carry):
        k = 2 * j
        wait_gather(rows0, gsem0)
        fire_write(k, rows0, wsem0)
        load_transform(k + 2, idx0, gidx0)
        wait_gather(rows1, gsem1)
        fire_write(k + 1, rows1, wsem1)
        load_transform(k + 3, idx1, gidx1)
        wait_write(k, rows0, wsem0)
        fire_gather(gidx0, rows0, gsem0)
        wait_write(k + 1, rows1, wsem1)
        fire_gather(gidx1, rows1, gsem1)
        return carry

    lax.fori_loop(0, _NBATCH // 2 - 1, pair, 0)

    k = _NBATCH - 2
    wait_gather(rows0, gsem0)
    fire_write(k, rows0, wsem0)
    wait_gather(rows1, gsem1)
    fire_write(k + 1, rows1, wsem1)
    wait_write(k, rows0, wsem0)
    wait_write(k + 1, rows1, wsem1)


@jax.jit
def _flat_gather(tab, idx, off):
    mesh = plsc.VectorSubcoreMesh(core_axis_name="c", subcore_axis_name="s")
    return pl.kernel(
        _gather_body,
        out_type=jax.ShapeDtypeStruct((_N, _EMBED), jnp.float32),
        mesh=mesh,
        compiler_params=pltpu.CompilerParams(use_tc_tiling_on_sc=False),
        scratch_types=[
            pltpu.VMEM((_BATCH,), jnp.int32),
            pltpu.VMEM((_BATCH,), jnp.int32),
            pltpu.VMEM((_BATCH,), jnp.int32),
            pltpu.VMEM((_BATCH,), jnp.int32),
            pltpu.VMEM((_BATCH, _EMBED), jnp.float32),
            pltpu.VMEM((_BATCH, _EMBED), jnp.float32),
            pltpu.VMEM((208,), jnp.int32),
            pltpu.SemaphoreType.DMA,
            pltpu.SemaphoreType.DMA,
            pltpu.SemaphoreType.DMA,
            pltpu.SemaphoreType.DMA,
        ],
    )(tab, idx, off)


def kernel(x, tables):
    tab = tables.reshape(_NUM_TOKENS * _VOCAB, _EMBED)
    idx = x.reshape(_N)
    out = _flat_gather(tab, idx, _OFF)
    return out.reshape(_B, _L, _NUM_TOKENS * _EMBED)
